# Initial kernel scaffold; baseline (speedup 1.0000x reference)
import jax
import jax.numpy as jnp
from jax.experimental import pallas as pl


def kernel(x, edge_index, W0, b0, g0, be0, W1, b1, g1, be1, W2, b2, g2, be2, W3, b3, g3, be3):
    raise NotImplementedError("write your pallas kernel here")


# speedup vs baseline: 3.2044x; 3.2044x over previous
import functools

import jax
import jax.numpy as jnp
from jax import lax
from jax.experimental import pallas as pl
from jax.experimental.pallas import tpu as pltpu
from jax.experimental.pallas import tpu_sc as plsc

N = 10000
E = 320000
H = 64
D = 128
EPS = 1e-5

NC = 2
NS = 16
NW = NC * NS
C = 128
NCHUNK = 80
EPAD = NW * NCHUNK * C
NP = 10240
RPT = NP // NS


def _seg_sum(table, eidx, zeros):
  mesh = plsc.VectorSubcoreMesh(core_axis_name="c", subcore_axis_name="s")

  @functools.partial(
      pl.kernel,
      out_type=jax.ShapeDtypeStruct((NC, NP, D), jnp.float32),
      mesh=mesh,
      scratch_types=[
          pltpu.VMEM((2, NCHUNK, C), jnp.int32),
          pltpu.VMEM((C, D), jnp.float32),
          pltpu.VMEM_SHARED((NP, D), jnp.float32),
          pltpu.SemaphoreType.DMA,
      ],
  )
  def seg_kernel(table_hbm, eidx_hbm, zeros_hbm, out_hbm,
                 sd_v, rows_v, acc, sem):
    cid = lax.axis_index("c")
    sid = lax.axis_index("s")
    w = cid * NS + sid

    pltpu.sync_copy(eidx_hbm.at[w], sd_v)
    pltpu.sync_copy(zeros_hbm.at[pl.ds(sid * RPT, RPT)],
                    acc.at[pl.ds(sid * RPT, RPT)])
    plsc.subcore_barrier()

    @pl.loop(0, NCHUNK)
    def _chunk(j):
      pltpu.async_copy(table_hbm.at[sd_v.at[0, j]], rows_v, sem).wait()
      pltpu.sync_copy(rows_v, acc.at[sd_v.at[1, j]], add=True)

    plsc.subcore_barrier()
    pltpu.sync_copy(acc.at[pl.ds(sid * RPT, RPT)],
                    out_hbm.at[cid, pl.ds(sid * RPT, RPT)])

  return seg_kernel(table, eidx, zeros)


def _bn(u, g, b):
  m = jnp.mean(u, axis=0, keepdims=True)
  d = u - m
  v = jnp.mean(d * d, axis=0, keepdims=True)
  return d * lax.rsqrt(v + EPS) * g + b


def _fused1_body(x_ref, s_ref, w0_ref, b0_ref, g0_ref, be0_ref, w1_ref,
                 b1_ref, g1_ref, be1_ref, o_ref):
  a = x_ref[...] + s_ref[0, :N] + s_ref[1, :N]
  u = jnp.dot(a, w0_ref[...], preferred_element_type=jnp.float32) + b0_ref[...]
  u = jnp.maximum(_bn(u, g0_ref[...], be0_ref[...]), 0.0)
  v = jnp.dot(u, w1_ref[...], preferred_element_type=jnp.float32) + b1_ref[...]
  h = jnp.maximum(_bn(v, g1_ref[...], be1_ref[...]), 0.0)
  o_ref[...] = jnp.concatenate([h, jnp.zeros_like(h)], axis=1)


def _fused2_body(h_ref, s_ref, w2_ref, b2_ref, g2_ref, be2_ref, w3_ref,
                 b3_ref, g3_ref, be3_ref, o_ref):
  a = h_ref[:, :H] + s_ref[0, :N, :H] + s_ref[1, :N, :H]
  p = jnp.dot(a, w2_ref[...], preferred_element_type=jnp.float32) + b2_ref[...]
  p = jnp.maximum(_bn(p, g2_ref[...], be2_ref[...]), 0.0)
  q = jnp.dot(p, w3_ref[...], preferred_element_type=jnp.float32) + b3_ref[...]
  o_ref[...] = _bn(q, g3_ref[...], be3_ref[...])


def kernel(x, edge_index, W0, b0, g0, be0, W1, b1, g1, be1,
           W2, b2, g2, be2, W3, b3, g3, be3):
  pad = EPAD - E
  src = jnp.concatenate([edge_index[0], jnp.zeros((pad,), jnp.int32)])
  dst = jnp.concatenate([edge_index[1], jnp.full((pad,), N, jnp.int32)])
  eidx = jnp.stack([src.reshape(NW, NCHUNK, C),
                    dst.reshape(NW, NCHUNK, C)], axis=1)
  zeros = jnp.zeros((NP, D), jnp.float32)
  r2 = lambda a: a.reshape(1, -1)

  s1 = _seg_sum(x, eidx, zeros)

  h = pl.pallas_call(
      _fused1_body,
      out_shape=jax.ShapeDtypeStruct((N, D), jnp.float32),
  )(x, s1, W0, r2(b0), r2(g0), r2(be0), W1, r2(b1), r2(g1), r2(be1))

  s2 = _seg_sum(h, eidx, zeros)

  out = pl.pallas_call(
      _fused2_body,
      out_shape=jax.ShapeDtypeStruct((N, D), jnp.float32),
  )(h, s2, W2, r2(b2), r2(g2), r2(be2), W3, r2(b3), r2(g3), r2(be3))

  return out
